# Initial kernel scaffold; baseline (speedup 1.0000x reference)
import jax
import jax.numpy as jnp
from jax.experimental import pallas as pl


def kernel(x, conv_w2dt, fc1_wb, bn_gamma, bn_beta, fc2_w, fc2_b):
    raise NotImplementedError("write your pallas kernel here")


# speedup vs baseline: 2.1974x; 2.1974x over previous
import functools

import jax
import jax.numpy as jnp
from jax.experimental import pallas as pl
from jax.experimental.pallas import tpu as pltpu

_BN_EPS = 1e-5
_P = 9
_K = 16 * _P
_TB = 4096


def _fused_body(n_vals, x_ref, wconv_ref, w1b1_ref, sc_ref, o_ref,
                conv_scr, mom_scr):
    t = pl.program_id(0)
    i = pl.program_id(1)

    @pl.when(t == 0)
    def _conv_phase():
        c = jax.lax.dot_general(
            wconv_ref[...], x_ref[...],
            dimension_numbers=(((1,), (1,)), ((), ())),
            preferred_element_type=jnp.float32)
        conv_scr[i] = c

        @pl.when(i == 0)
        def _init_moments():
            mom_scr[...] = jnp.zeros_like(mom_scr)

        mom_scr[0:1, 0:1] += jnp.sum(c, keepdims=True)
        mom_scr[0:1, 1:2] += jnp.sum(c * c, keepdims=True)

    @pl.when(t == 1)
    def _head_phase():
        inv_n = 1.0 / jnp.float32(n_vals)
        mean = mom_scr[0:1, 0:1] * inv_n
        var = mom_scr[0:1, 1:2] * inv_n - mean * mean
        scale = sc_ref[0] * jax.lax.rsqrt(var + _BN_EPS)
        shift = sc_ref[1] - mean * scale
        y = jnp.maximum(conv_scr[i] * scale + shift, 0.0)
        h = jax.lax.dot_general(
            w1b1_ref[:, 0:_P], y,
            dimension_numbers=(((1,), (0,)), ((), ())),
            preferred_element_type=jnp.float32) + w1b1_ref[:, _P:_P + 1]
        h = jnp.maximum(h, 0.0)
        v = (sc_ref[2] * h[0:1, :] + sc_ref[3] * h[1:2, :]
             + sc_ref[4] * h[2:3, :] + sc_ref[5])
        o_ref[...] = jnp.tanh(v)


def kernel(x, conv_w2dt, fc1_wb, bn_gamma, bn_beta, fc2_w, fc2_b):
    B = x.shape[0]
    xf = x.reshape(B, _K)

    tb = min(_TB, -(-B // 128) * 128)
    bp = -(-B // tb) * tb
    if bp != B:
        xf = jnp.pad(xf, ((0, bp - B), (0, 0)))
    g = bp // tb

    sc = jnp.concatenate(
        [bn_gamma, bn_beta, fc2_w[0], fc2_b]).astype(jnp.float32)

    out = pl.pallas_call(
        functools.partial(_fused_body, B * _P),
        grid=(2, g),
        in_specs=[
            pl.BlockSpec((tb, _K), lambda t, i: ((1 - t) * i, 0)),
            pl.BlockSpec((_P, _K), lambda t, i: (0, 0)),
            pl.BlockSpec((3, _P + 1), lambda t, i: (0, 0)),
            pl.BlockSpec(memory_space=pltpu.MemorySpace.SMEM),
        ],
        out_specs=pl.BlockSpec((1, tb), lambda t, i: (0, t * i)),
        out_shape=jax.ShapeDtypeStruct((1, bp), jnp.float32),
        scratch_shapes=[
            pltpu.VMEM((g, _P, tb), jnp.float32),
            pltpu.VMEM((1, 2), jnp.float32),
        ],
        compiler_params=pltpu.CompilerParams(
            dimension_semantics=("arbitrary", "arbitrary")),
    )(xf, conv_w2dt, fc1_wb, sc)

    return out[0, :B].reshape(B, 1)


# speedup vs baseline: 15.6423x; 7.1185x over previous
import functools

import jax
import jax.numpy as jnp
from jax.experimental import pallas as pl
from jax.experimental.pallas import tpu as pltpu

_BN_EPS = 1e-5
_P = 9
_C = 16
_K = _C * _P
_TB = 4096


def _fused_body(n_vals, xt_ref, wconv_ref, w1b1_ref, gam_ref, bet_ref,
                w2_ref, b2_ref, o_ref, conv_scr, mom_scr):
    t = pl.program_id(0)
    i = pl.program_id(1)

    @pl.when(t == 0)
    def _conv_phase():
        c = jax.lax.dot_general(
            wconv_ref[...], xt_ref[...],
            dimension_numbers=(((1,), (0,)), ((), ())),
            preferred_element_type=jnp.float32)
        conv_scr[i] = c

        @pl.when(i == 0)
        def _init_moments():
            mom_scr[...] = jnp.zeros_like(mom_scr)

        mom_scr[0:1, 0:1] += jnp.sum(c, keepdims=True)
        mom_scr[0:1, 1:2] += jnp.sum(c * c, keepdims=True)

    @pl.when(t == 1)
    def _head_phase():
        inv_n = 1.0 / jnp.float32(n_vals)
        mean = mom_scr[0:1, 0:1] * inv_n
        var = mom_scr[0:1, 1:2] * inv_n - mean * mean
        scale = gam_ref[0] * jax.lax.rsqrt(var + _BN_EPS)
        shift = bet_ref[0] - mean * scale
        y = jnp.maximum(conv_scr[i] * scale + shift, 0.0)
        h = jax.lax.dot_general(
            w1b1_ref[:, 0:_P], y,
            dimension_numbers=(((1,), (0,)), ((), ())),
            preferred_element_type=jnp.float32) + w1b1_ref[:, _P:_P + 1]
        h = jnp.maximum(h, 0.0)
        v = (w2_ref[0, 0] * h[0:1, :] + w2_ref[0, 1] * h[1:2, :]
             + w2_ref[0, 2] * h[2:3, :] + b2_ref[0])
        o_ref[...] = jnp.tanh(v)[0]


def kernel(x, conv_w2dt, fc1_wb, bn_gamma, bn_beta, fc2_w, fc2_b):
    B = x.shape[0]
    xt = jnp.transpose(x, (2, 3, 1, 0)).reshape(_K, B)
    wconv = jnp.transpose(
        conv_w2dt.reshape(_P, _C, _P), (0, 2, 1)).reshape(_P, _K)

    tb = min(_TB, -(-B // 1024) * 1024)
    bp = -(-B // tb) * tb
    if bp != B:
        xt = jnp.pad(xt, ((0, 0), (0, bp - B)))
    g = bp // tb

    smem = pl.BlockSpec(memory_space=pltpu.MemorySpace.SMEM)
    out = pl.pallas_call(
        functools.partial(_fused_body, B * _P),
        grid=(2, g),
        in_specs=[
            pl.BlockSpec((_K, tb), lambda t, i: (0, (1 - t) * i)),
            pl.BlockSpec((_P, _K), lambda t, i: (0, 0)),
            pl.BlockSpec((3, _P + 1), lambda t, i: (0, 0)),
            smem, smem, smem, smem,
        ],
        out_specs=pl.BlockSpec((tb,), lambda t, i: (t * i,)),
        out_shape=jax.ShapeDtypeStruct((bp,), jnp.float32),
        scratch_shapes=[
            pltpu.VMEM((g, _P, tb), jnp.float32),
            pltpu.VMEM((1, 2), jnp.float32),
        ],
        compiler_params=pltpu.CompilerParams(
            dimension_semantics=("arbitrary", "arbitrary")),
    )(xt, wconv, fc1_wb, bn_gamma, bn_beta, fc2_w, fc2_b)

    return out[:B].reshape(B, 1)


# speedup vs baseline: 21.2884x; 1.3609x over previous
import functools

import jax
import jax.numpy as jnp
from jax.experimental import pallas as pl
from jax.experimental.pallas import tpu as pltpu

_BN_EPS = 1e-5
_P = 9
_C = 16
_K = _C * _P
_TB = 8192


def _fused_body(n_vals, xt_ref, wconv_ref, w1b1_ref, gam_ref, bet_ref,
                w2_ref, b2_ref, o_ref, conv_scr, mom_scr):
    t = pl.program_id(0)
    i = pl.program_id(1)

    @pl.when(t == 0)
    def _conv_phase():
        c = jax.lax.dot_general(
            wconv_ref[...], xt_ref[...],
            dimension_numbers=(((1,), (0,)), ((), ())),
            preferred_element_type=jnp.float32)
        conv_scr[i] = c

        @pl.when(i == 0)
        def _init_moments():
            mom_scr[...] = jnp.zeros_like(mom_scr)

        mom_scr[0:1, 0:1] += jnp.sum(c, keepdims=True)
        mom_scr[0:1, 1:2] += jnp.sum(c * c, keepdims=True)

    @pl.when(t == 1)
    def _head_phase():
        inv_n = 1.0 / jnp.float32(n_vals)
        mean = mom_scr[0:1, 0:1] * inv_n
        var = mom_scr[0:1, 1:2] * inv_n - mean * mean
        scale = gam_ref[0] * jax.lax.rsqrt(var + _BN_EPS)
        shift = bet_ref[0] - mean * scale
        y = jnp.maximum(conv_scr[i] * scale + shift, 0.0)
        h = jax.lax.dot_general(
            w1b1_ref[:, 0:_P], y,
            dimension_numbers=(((1,), (0,)), ((), ())),
            preferred_element_type=jnp.float32) + w1b1_ref[:, _P:_P + 1]
        h = jnp.maximum(h, 0.0)
        v = (w2_ref[0, 0] * h[0:1, :] + w2_ref[0, 1] * h[1:2, :]
             + w2_ref[0, 2] * h[2:3, :] + b2_ref[0])
        o_ref[...] = jnp.tanh(v)[0]


def kernel(x, conv_w2dt, fc1_wb, bn_gamma, bn_beta, fc2_w, fc2_b):
    B = x.shape[0]
    xt = jnp.transpose(x, (2, 3, 1, 0)).reshape(_K, B)
    wconv = jnp.transpose(
        conv_w2dt.reshape(_P, _C, _P), (0, 2, 1)).reshape(_P, _K)

    tb = min(_TB, -(-B // 1024) * 1024)
    bp = -(-B // tb) * tb
    if bp != B:
        xt = jnp.pad(xt, ((0, 0), (0, bp - B)))
    g = bp // tb

    smem = pl.BlockSpec(memory_space=pltpu.MemorySpace.SMEM)
    out = pl.pallas_call(
        functools.partial(_fused_body, B * _P),
        grid=(2, g),
        in_specs=[
            pl.BlockSpec((_K, tb), lambda t, i: (0, (1 - t) * i)),
            pl.BlockSpec((_P, _K), lambda t, i: (0, 0)),
            pl.BlockSpec((3, _P + 1), lambda t, i: (0, 0)),
            smem, smem, smem, smem,
        ],
        out_specs=pl.BlockSpec((tb,), lambda t, i: (t * i,)),
        out_shape=jax.ShapeDtypeStruct((bp,), jnp.float32),
        scratch_shapes=[
            pltpu.VMEM((g, _P, tb), jnp.float32),
            pltpu.VMEM((1, 2), jnp.float32),
        ],
        compiler_params=pltpu.CompilerParams(
            dimension_semantics=("arbitrary", "arbitrary")),
    )(xt, wconv, fc1_wb, bn_gamma, bn_beta, fc2_w, fc2_b)

    return out[:B].reshape(B, 1)


# speedup vs baseline: 22.9798x; 1.0795x over previous
import functools

import jax
import jax.numpy as jnp
from jax.experimental import pallas as pl
from jax.experimental.pallas import tpu as pltpu

_BN_EPS = 1e-5
_P = 9
_C = 16
_K = _C * _P
_TB = 16384


def _fused_body(n_vals, xt_ref, wconv_ref, w1b1_ref, gam_ref, bet_ref,
                w2_ref, b2_ref, o_ref, conv_scr, mom_scr):
    t = pl.program_id(0)
    i = pl.program_id(1)

    @pl.when(t == 0)
    def _conv_phase():
        c = jax.lax.dot_general(
            wconv_ref[...], xt_ref[...],
            dimension_numbers=(((1,), (0,)), ((), ())),
            preferred_element_type=jnp.float32)
        conv_scr[i] = c

        @pl.when(i == 0)
        def _init_moments():
            mom_scr[...] = jnp.zeros_like(mom_scr)

        mom_scr[0:1, 0:1] += jnp.sum(c, keepdims=True)
        mom_scr[0:1, 1:2] += jnp.sum(c * c, keepdims=True)

    @pl.when(t == 1)
    def _head_phase():
        inv_n = 1.0 / jnp.float32(n_vals)
        mean = mom_scr[0:1, 0:1] * inv_n
        var = mom_scr[0:1, 1:2] * inv_n - mean * mean
        scale = gam_ref[0] * jax.lax.rsqrt(var + _BN_EPS)
        shift = bet_ref[0] - mean * scale
        y = jnp.maximum(conv_scr[i] * scale + shift, 0.0)
        h = jax.lax.dot_general(
            w1b1_ref[:, 0:_P], y,
            dimension_numbers=(((1,), (0,)), ((), ())),
            preferred_element_type=jnp.float32) + w1b1_ref[:, _P:_P + 1]
        h = jnp.maximum(h, 0.0)
        v = (w2_ref[0, 0] * h[0:1, :] + w2_ref[0, 1] * h[1:2, :]
             + w2_ref[0, 2] * h[2:3, :] + b2_ref[0])
        o_ref[...] = jnp.tanh(v)[0]


def kernel(x, conv_w2dt, fc1_wb, bn_gamma, bn_beta, fc2_w, fc2_b):
    B = x.shape[0]
    xt = jnp.transpose(x, (2, 3, 1, 0)).reshape(_K, B)
    wconv = jnp.transpose(
        conv_w2dt.reshape(_P, _C, _P), (0, 2, 1)).reshape(_P, _K)

    tb = min(_TB, -(-B // 1024) * 1024)
    bp = -(-B // tb) * tb
    if bp != B:
        xt = jnp.pad(xt, ((0, 0), (0, bp - B)))
    g = bp // tb

    smem = pl.BlockSpec(memory_space=pltpu.MemorySpace.SMEM)
    out = pl.pallas_call(
        functools.partial(_fused_body, B * _P),
        grid=(2, g),
        in_specs=[
            pl.BlockSpec((_K, tb), lambda t, i: (0, (1 - t) * i)),
            pl.BlockSpec((_P, _K), lambda t, i: (0, 0)),
            pl.BlockSpec((3, _P + 1), lambda t, i: (0, 0)),
            smem, smem, smem, smem,
        ],
        out_specs=pl.BlockSpec((tb,), lambda t, i: (t * i,)),
        out_shape=jax.ShapeDtypeStruct((bp,), jnp.float32),
        scratch_shapes=[
            pltpu.VMEM((g, _P, tb), jnp.float32),
            pltpu.VMEM((1, 2), jnp.float32),
        ],
        compiler_params=pltpu.CompilerParams(
            dimension_semantics=("arbitrary", "arbitrary")),
    )(xt, wconv, fc1_wb, bn_gamma, bn_beta, fc2_w, fc2_b)

    return out[:B].reshape(B, 1)


# speedup vs baseline: 24.6935x; 1.0746x over previous
import functools

import jax
import jax.numpy as jnp
from jax.experimental import pallas as pl
from jax.experimental.pallas import tpu as pltpu

_BN_EPS = 1e-5
_P = 9
_C = 16
_K = _C * _P
_TB = 16384


def _fused_body(n_vals, xt_ref, wconv_ref, w1b1_ref, gam_ref, bet_ref,
                w2_ref, b2_ref, o_ref, conv_scr, mom_scr):
    t = pl.program_id(0)
    i = pl.program_id(1)

    @pl.when(t == 0)
    def _conv_phase():
        c = jax.lax.dot_general(
            wconv_ref[...], xt_ref[...],
            dimension_numbers=(((1,), (0,)), ((), ())),
            preferred_element_type=jnp.float32)
        conv_scr[i] = c

        @pl.when(i == 0)
        def _init_moments():
            mom_scr[...] = jnp.zeros_like(mom_scr)

        mom_scr[0:1, 0:1] += jnp.sum(c, keepdims=True)
        mom_scr[0:1, 1:2] += jnp.sum(c * c, keepdims=True)

    @pl.when(t == 1)
    def _head_phase():
        inv_n = 1.0 / jnp.float32(n_vals)
        mean = mom_scr[0:1, 0:1] * inv_n
        var = mom_scr[0:1, 1:2] * inv_n - mean * mean
        scale = gam_ref[0] * jax.lax.rsqrt(var + _BN_EPS)
        shift = bet_ref[0] - mean * scale
        y = jnp.maximum(conv_scr[i] * scale + shift, 0.0)
        h = jax.lax.dot_general(
            w1b1_ref[:, 0:_P], y,
            dimension_numbers=(((1,), (0,)), ((), ())),
            preferred_element_type=jnp.float32) + w1b1_ref[:, _P:_P + 1]
        h = jnp.maximum(h, 0.0)
        v = (w2_ref[0, 0] * h[0:1, :] + w2_ref[0, 1] * h[1:2, :]
             + w2_ref[0, 2] * h[2:3, :] + b2_ref[0])
        o_ref[...] = jnp.tanh(v)[0]


def kernel(x, conv_w2dt, fc1_wb, bn_gamma, bn_beta, fc2_w, fc2_b):
    B = x.shape[0]
    xt = jnp.transpose(x, (2, 3, 1, 0)).reshape(_K, B)
    wconv = jnp.transpose(
        conv_w2dt.reshape(_P, _C, _P), (0, 2, 1)).reshape(_P, _K)

    tb = min(_TB, -(-B // 1024) * 1024)
    bp = -(-B // tb) * tb
    if bp != B:
        xt = jnp.pad(xt, ((0, 0), (0, bp - B)))
    g = bp // tb

    smem = pl.BlockSpec(memory_space=pltpu.MemorySpace.SMEM)
    out = pl.pallas_call(
        functools.partial(_fused_body, B * _P),
        grid=(2, g),
        in_specs=[
            pl.BlockSpec((_K, tb), lambda t, i: (0, (1 - t) * i + t * (g - 1))),
            pl.BlockSpec((_P, _K), lambda t, i: (0, 0)),
            pl.BlockSpec((3, _P + 1), lambda t, i: (0, 0)),
            smem, smem, smem, smem,
        ],
        out_specs=pl.BlockSpec((tb,), lambda t, i: (t * i,)),
        out_shape=jax.ShapeDtypeStruct((bp,), jnp.float32),
        scratch_shapes=[
            pltpu.VMEM((g, _P, tb), jnp.float32),
            pltpu.VMEM((1, 2), jnp.float32),
        ],
        compiler_params=pltpu.CompilerParams(
            dimension_semantics=("arbitrary", "arbitrary")),
    )(xt, wconv, fc1_wb, bn_gamma, bn_beta, fc2_w, fc2_b)

    return out[:B].reshape(B, 1)


# speedup vs baseline: 26.7944x; 1.0851x over previous
import functools

import jax
import jax.numpy as jnp
import numpy as np
from jax.experimental import pallas as pl
from jax.experimental.pallas import tpu as pltpu

_BN_EPS = 1e-5
_P = 9
_C = 16
_K = _C * _P
_TB = 16384

_PERM = np.zeros((_K, _K), np.float32)
for _c in range(_C):
    for _p in range(_P):
        _PERM[_c * _P + _p, _p * _C + _c] = 1.0


def _fused_body(n_vals, xt_ref, wraw_ref, pm_ref, w1b1_ref, gam_ref, bet_ref,
                w2_ref, b2_ref, o_ref, conv_scr, mom_scr):
    t = pl.program_id(0)
    i = pl.program_id(1)

    @pl.when(t == 0)
    def _conv_phase():
        wconv = jax.lax.dot_general(
            wraw_ref[...], pm_ref[...],
            dimension_numbers=(((1,), (0,)), ((), ())),
            preferred_element_type=jnp.float32)
        c = jax.lax.dot_general(
            wconv, xt_ref[...],
            dimension_numbers=(((1,), (0,)), ((), ())),
            preferred_element_type=jnp.float32)
        conv_scr[i] = c

        @pl.when(i == 0)
        def _init_moments():
            mom_scr[...] = jnp.zeros_like(mom_scr)

        mom_scr[0:1, 0:1] += jnp.sum(c, keepdims=True)
        mom_scr[0:1, 1:2] += jnp.sum(c * c, keepdims=True)

    @pl.when(t == 1)
    def _head_phase():
        inv_n = 1.0 / jnp.float32(n_vals)
        mean = mom_scr[0:1, 0:1] * inv_n
        var = mom_scr[0:1, 1:2] * inv_n - mean * mean
        scale = gam_ref[0] * jax.lax.rsqrt(var + _BN_EPS)
        shift = bet_ref[0] - mean * scale
        y = jnp.maximum(conv_scr[i] * scale + shift, 0.0)
        h = jax.lax.dot_general(
            w1b1_ref[:, 0:_P], y,
            dimension_numbers=(((1,), (0,)), ((), ())),
            preferred_element_type=jnp.float32) + w1b1_ref[:, _P:_P + 1]
        h = jnp.maximum(h, 0.0)
        v = (w2_ref[0, 0] * h[0:1, :] + w2_ref[0, 1] * h[1:2, :]
             + w2_ref[0, 2] * h[2:3, :] + b2_ref[0])
        o_ref[...] = jnp.tanh(v)[0]


def kernel(x, conv_w2dt, fc1_wb, bn_gamma, bn_beta, fc2_w, fc2_b):
    B = x.shape[0]
    xt = jnp.transpose(x, (2, 3, 1, 0)).reshape(_K, B)

    tb = min(_TB, -(-B // 1024) * 1024)
    bp = -(-B // tb) * tb
    if bp != B:
        xt = jnp.pad(xt, ((0, 0), (0, bp - B)))
    g = bp // tb

    smem = pl.BlockSpec(memory_space=pltpu.MemorySpace.SMEM)
    out = pl.pallas_call(
        functools.partial(_fused_body, B * _P),
        grid=(2, g),
        in_specs=[
            pl.BlockSpec((_K, tb), lambda t, i: (0, (1 - t) * i + t * (g - 1))),
            pl.BlockSpec((_P, _K), lambda t, i: (0, 0)),
            pl.BlockSpec((_K, _K), lambda t, i: (0, 0)),
            pl.BlockSpec((3, _P + 1), lambda t, i: (0, 0)),
            smem, smem, smem, smem,
        ],
        out_specs=pl.BlockSpec((tb,), lambda t, i: (t * i,)),
        out_shape=jax.ShapeDtypeStruct((bp,), jnp.float32),
        scratch_shapes=[
            pltpu.VMEM((g, _P, tb), jnp.float32),
            pltpu.VMEM((1, 2), jnp.float32),
        ],
        compiler_params=pltpu.CompilerParams(
            dimension_semantics=("arbitrary", "arbitrary")),
    )(xt, conv_w2dt, jnp.asarray(_PERM), fc1_wb, bn_gamma, bn_beta,
      fc2_w, fc2_b)

    return out[:B].reshape(B, 1)


# speedup vs baseline: 27.0973x; 1.0113x over previous
import functools

import jax
import jax.numpy as jnp
import numpy as np
from jax.experimental import pallas as pl
from jax.experimental.pallas import tpu as pltpu

_BN_EPS = 1e-5
_P = 9
_C = 16
_K = _C * _P
_TB = 32768

_PERM = np.zeros((_K, _K), np.float32)
for _c in range(_C):
    for _p in range(_P):
        _PERM[_c * _P + _p, _p * _C + _c] = 1.0


def _fused_body(n_vals, xt_ref, wraw_ref, pm_ref, w1b1_ref, gam_ref, bet_ref,
                w2_ref, b2_ref, o_ref, conv_scr, mom_scr):
    t = pl.program_id(0)
    i = pl.program_id(1)

    @pl.when(t == 0)
    def _conv_phase():
        wconv = jax.lax.dot_general(
            wraw_ref[...], pm_ref[...],
            dimension_numbers=(((1,), (0,)), ((), ())),
            preferred_element_type=jnp.float32)
        c = jax.lax.dot_general(
            wconv, xt_ref[...],
            dimension_numbers=(((1,), (0,)), ((), ())),
            preferred_element_type=jnp.float32)
        conv_scr[i] = c

        @pl.when(i == 0)
        def _init_moments():
            mom_scr[...] = jnp.zeros_like(mom_scr)

        mom_scr[0:1, 0:1] += jnp.sum(c, keepdims=True)
        mom_scr[0:1, 1:2] += jnp.sum(c * c, keepdims=True)

    @pl.when(t == 1)
    def _head_phase():
        inv_n = 1.0 / jnp.float32(n_vals)
        mean = mom_scr[0:1, 0:1] * inv_n
        var = mom_scr[0:1, 1:2] * inv_n - mean * mean
        scale = gam_ref[0] * jax.lax.rsqrt(var + _BN_EPS)
        shift = bet_ref[0] - mean * scale
        y = jnp.maximum(conv_scr[i] * scale + shift, 0.0)
        h = jax.lax.dot_general(
            w1b1_ref[:, 0:_P], y,
            dimension_numbers=(((1,), (0,)), ((), ())),
            preferred_element_type=jnp.float32) + w1b1_ref[:, _P:_P + 1]
        h = jnp.maximum(h, 0.0)
        v = (w2_ref[0, 0] * h[0:1, :] + w2_ref[0, 1] * h[1:2, :]
             + w2_ref[0, 2] * h[2:3, :] + b2_ref[0])
        o_ref[...] = jnp.tanh(v)[0]


def kernel(x, conv_w2dt, fc1_wb, bn_gamma, bn_beta, fc2_w, fc2_b):
    B = x.shape[0]
    xt = jnp.transpose(x, (2, 3, 1, 0)).reshape(_K, B)

    tb = min(_TB, -(-B // 1024) * 1024)
    bp = -(-B // tb) * tb
    if bp != B:
        xt = jnp.pad(xt, ((0, 0), (0, bp - B)))
    g = bp // tb

    smem = pl.BlockSpec(memory_space=pltpu.MemorySpace.SMEM)
    out = pl.pallas_call(
        functools.partial(_fused_body, B * _P),
        grid=(2, g),
        in_specs=[
            pl.BlockSpec((_K, tb), lambda t, i: (0, (1 - t) * i + t * (g - 1))),
            pl.BlockSpec((_P, _K), lambda t, i: (0, 0)),
            pl.BlockSpec((_K, _K), lambda t, i: (0, 0)),
            pl.BlockSpec((3, _P + 1), lambda t, i: (0, 0)),
            smem, smem, smem, smem,
        ],
        out_specs=pl.BlockSpec((tb,), lambda t, i: (t * i,)),
        out_shape=jax.ShapeDtypeStruct((bp,), jnp.float32),
        scratch_shapes=[
            pltpu.VMEM((g, _P, tb), jnp.float32),
            pltpu.VMEM((1, 2), jnp.float32),
        ],
        compiler_params=pltpu.CompilerParams(
            dimension_semantics=("arbitrary", "arbitrary")),
    )(xt, conv_w2dt, jnp.asarray(_PERM), fc1_wb, bn_gamma, bn_beta,
      fc2_w, fc2_b)

    return out[:B].reshape(B, 1)
